# baseline (speedup 1.0000x reference)
import functools

import jax
import jax.numpy as jnp
from jax import lax
from jax.experimental import pallas as pl
from jax.experimental.pallas import tpu as pltpu
from jax.experimental.pallas import tpu_sc as plsc

N = 1_600_000
G = 10_000
EPS = 1e-08

NC = 2
NS = 16
L = 16
NW = NC * NS
APW = N // NW
CS = 10_000
NCHUNK = APW // CS
GP = 10_240
GPW = GP // NW
UNROLL = 5
EU = 25

_mesh = plsc.VectorSubcoreMesh(core_axis_name="c", subcore_axis_name="s")
_params = pltpu.CompilerParams(
    needs_layout_passes=False, use_tc_tiling_on_sc=False
)


def _wid():
    return lax.axis_index("s") * NC + lax.axis_index("c")


def _exp_loop(lbuf):
    @plsc.parallel_loop(0, CS // L, unroll=EU)
    def _(i):
        s = pl.ds(i * L, L)
        lbuf[s] = jnp.exp(lbuf[s])


def _p1_body(ch_hbm, lv_hbm, bi_hbm, part_hbm,
             cb0, lb0, ib0, cb1, lb1, ib1, acc2, rbuf, shared, sm0, sm1):
    cid = lax.axis_index("c")
    sid = lax.axis_index("s")
    wid = sid * NC + cid
    base = wid * APW
    bufs = ((cb0, lb0, ib0, sm0), (cb1, lb1, ib1, sm1))
    accc = acc2.at[0]
    accv = acc2.at[1]

    def issue(k):
        cb, lb, ib, sem = bufs[k % 2]
        off = base + k * CS
        return (pltpu.async_copy(ch_hbm.at[pl.ds(off, CS)], cb, sem),
                pltpu.async_copy(lv_hbm.at[pl.ds(off, CS)], lb, sem),
                pltpu.async_copy(bi_hbm.at[pl.ds(off, CS)], ib, sem))

    pending = issue(0)

    @plsc.parallel_loop(0, GP // L, unroll=8)
    def _(i):
        s = pl.ds(i * L, L)
        accc[s] = jnp.zeros((L,), jnp.float32)
        accv[s] = jnp.zeros((L,), jnp.float32)

    lanes = lax.iota(jnp.int32, L) * (CS // L)

    for k in range(NCHUNK):
        for cp in pending:
            cp.wait()
        cb, lb, ib, _ = bufs[k % 2]
        if k + 1 < NCHUNK:
            pending = issue(k + 1)
        _exp_loop(lb)

        @plsc.parallel_loop(0, CS // L, unroll=UNROLL)
        def _(i):
            pos = lanes + i
            idx = plsc.load_gather(ib, [pos])
            c = plsc.load_gather(cb, [pos])
            v = plsc.load_gather(lb, [pos])
            plsc.addupdate_scatter(accc, [idx], c)
            plsc.addupdate_scatter(accv, [idx], v)

    pltpu.sync_copy(acc2, shared.at[pl.ds(2 * sid, 2)])
    plsc.subcore_barrier()
    GSL = GP // NS
    goff = sid * GSL
    pltpu.sync_copy(shared.at[:, pl.ds(goff, GSL)], rbuf)

    @plsc.parallel_loop(0, GSL // L, unroll=4)
    def _(j):
        s = pl.ds(j * L, L)
        cs = rbuf[0, s]
        vs = rbuf[1, s]
        for tt in range(1, NS):
            cs = cs + rbuf[2 * tt, s]
            vs = vs + rbuf[2 * tt + 1, s]
        rbuf[0, s] = cs
        rbuf[1, s] = vs

    pltpu.sync_copy(rbuf.at[pl.ds(0, 2)], part_hbm.at[cid, :, pl.ds(goff, GSL)])


_pass1 = functools.partial(
    pl.kernel,
    mesh=_mesh,
    compiler_params=_params,
    out_type=jax.ShapeDtypeStruct((NC, 2, GP), jnp.float32),
    scratch_types=[
        pltpu.VMEM((CS,), jnp.float32),
        pltpu.VMEM((CS,), jnp.float32),
        pltpu.VMEM((CS,), jnp.int32),
        pltpu.VMEM((CS,), jnp.float32),
        pltpu.VMEM((CS,), jnp.float32),
        pltpu.VMEM((CS,), jnp.int32),
        pltpu.VMEM((2, GP), jnp.float32),
        pltpu.VMEM((2 * NS, GP // NS), jnp.float32),
        pltpu.VMEM_SHARED((2 * NS, GP), jnp.float32),
        pltpu.SemaphoreType.DMA,
        pltpu.SemaphoreType.DMA,
    ],
)(_p1_body)


def _p3_body(ch_hbm, lv_hbm, bi_hbm, part_hbm, formal_hbm, out_hbm,
             cb0, lb0, ib0, cb1, lb1, ib1, sbuf, pbuf, smp, sm0, sm1):
    wid = _wid()
    base = wid * APW
    bufs = ((cb0, lb0, ib0, sm0), (cb1, lb1, ib1, sm1))

    def issue(k):
        cb, lb, ib, sem = bufs[k % 2]
        off = base + k * CS
        return (pltpu.async_copy(ch_hbm.at[pl.ds(off, CS)], cb, sem),
                pltpu.async_copy(lv_hbm.at[pl.ds(off, CS)], lb, sem),
                pltpu.async_copy(bi_hbm.at[pl.ds(off, CS)], ib, sem))

    part_cp = pltpu.async_copy(part_hbm, pbuf, smp)
    formal_cp = pltpu.async_copy(formal_hbm, sbuf, smp)
    pending = issue(0)
    part_cp.wait()
    formal_cp.wait()

    @plsc.parallel_loop(0, GP // L, unroll=8)
    def _(j):
        s = pl.ds(j * L, L)
        cs = pbuf[0, 0, s] + pbuf[1, 0, s]
        vs = pbuf[0, 1, s] + pbuf[1, 1, s]
        sbuf[s] = (sbuf[s] - cs) / (vs + EPS)

    writeback = [None, None]

    for k in range(NCHUNK):
        for cp in pending:
            cp.wait()
        cb, lb, ib, sem = bufs[k % 2]
        if k + 1 < NCHUNK:
            wb = writeback[(k + 1) % 2]
            if wb is not None:
                wb.wait()
            pending = issue(k + 1)
        _exp_loop(lb)

        @plsc.parallel_loop(0, CS // L, unroll=UNROLL)
        def _(i):
            s = pl.ds(i * L, L)
            w = plsc.load_gather(sbuf, [ib[s]])
            cb[s] = cb[s] + lb[s] * w

        off = base + k * CS
        writeback[k % 2] = pltpu.async_copy(cb, out_hbm.at[pl.ds(off, CS)], sem)

    for wb in writeback:
        if wb is not None:
            wb.wait()


_pass3 = functools.partial(
    pl.kernel,
    mesh=_mesh,
    compiler_params=_params,
    out_type=jax.ShapeDtypeStruct((N,), jnp.float32),
    scratch_types=[
        pltpu.VMEM((CS,), jnp.float32),
        pltpu.VMEM((CS,), jnp.float32),
        pltpu.VMEM((CS,), jnp.int32),
        pltpu.VMEM((CS,), jnp.float32),
        pltpu.VMEM((CS,), jnp.float32),
        pltpu.VMEM((CS,), jnp.int32),
        pltpu.VMEM((GP,), jnp.float32),
        pltpu.VMEM((NC, 2, GP), jnp.float32),
        pltpu.SemaphoreType.DMA,
        pltpu.SemaphoreType.DMA,
        pltpu.SemaphoreType.DMA,
    ],
)(_p3_body)


def kernel(charges, log_variance, batch_index, formal_charges):
    partials = _pass1(charges, log_variance, batch_index)
    formal_pad = jnp.pad(formal_charges.astype(jnp.float32), (0, GP - G))
    return _pass3(charges, log_variance, batch_index, partials, formal_pad)


# speedup vs baseline: 1.0033x; 1.0033x over previous
import functools

import jax
import jax.numpy as jnp
from jax import lax
from jax.experimental import pallas as pl
from jax.experimental.pallas import tpu as pltpu
from jax.experimental.pallas import tpu_sc as plsc

N = 1_600_000
G = 10_000
EPS = 1e-08

NC = 2
NS = 16
L = 16
NW = NC * NS
APW = N // NW
CS = 10_000
NCHUNK = APW // CS
GP = 10_240
GPW = GP // NW
UNROLL = 5
EU = 25

_mesh = plsc.VectorSubcoreMesh(core_axis_name="c", subcore_axis_name="s")
_params = pltpu.CompilerParams(
    needs_layout_passes=False, use_tc_tiling_on_sc=False
)


def _wid():
    return lax.axis_index("s") * NC + lax.axis_index("c")


def _exp_loop(lbuf):
    @plsc.parallel_loop(0, CS // L, unroll=EU)
    def _(i):
        s = pl.ds(i * L, L)
        lbuf[s] = jnp.exp(lbuf[s])


def _p1_body(ch_hbm, lv_hbm, bi_hbm, part_hbm,
             cb0, lb0, ib0, cb1, lb1, ib1, acc2, rbuf, shared, sm0, sm1):
    cid = lax.axis_index("c")
    sid = lax.axis_index("s")
    wid = sid * NC + cid
    base = wid * APW
    bufs = ((cb0, lb0, ib0, sm0), (cb1, lb1, ib1, sm1))
    accc = acc2.at[0]
    accv = acc2.at[1]

    def issue(k):
        cb, lb, ib, sem = bufs[k % 2]
        off = base + k * CS
        return (pltpu.async_copy(ch_hbm.at[pl.ds(off, CS)], cb, sem),
                pltpu.async_copy(lv_hbm.at[pl.ds(off, CS)], lb, sem),
                pltpu.async_copy(bi_hbm.at[pl.ds(off, CS)], ib, sem))

    pending = issue(0)

    @plsc.parallel_loop(0, GP // L, unroll=8)
    def _(i):
        s = pl.ds(i * L, L)
        accc[s] = jnp.zeros((L,), jnp.float32)
        accv[s] = jnp.zeros((L,), jnp.float32)

    lanes = lax.iota(jnp.int32, L) * (CS // L)

    for k in range(NCHUNK):
        for cp in pending:
            cp.wait()
        cb, lb, ib, _ = bufs[k % 2]
        if k + 1 < NCHUNK:
            pending = issue(k + 1)
        _exp_loop(lb)

        @plsc.parallel_loop(0, CS // L, unroll=UNROLL)
        def _(i):
            pos = lanes + i
            idx = plsc.load_gather(ib, [pos])
            c = plsc.load_gather(cb, [pos])
            v = plsc.load_gather(lb, [pos])
            plsc.addupdate_scatter(accc, [idx], c)
            plsc.addupdate_scatter(accv, [idx], v)

    pltpu.sync_copy(acc2, shared.at[pl.ds(2 * sid, 2)])
    plsc.subcore_barrier()
    GSL = GP // NS
    goff = sid * GSL
    pltpu.sync_copy(shared.at[:, pl.ds(goff, GSL)], rbuf)

    @plsc.parallel_loop(0, GSL // L, unroll=4)
    def _(j):
        s = pl.ds(j * L, L)
        cs = rbuf[0, s]
        vs = rbuf[1, s]
        for tt in range(1, NS):
            cs = cs + rbuf[2 * tt, s]
            vs = vs + rbuf[2 * tt + 1, s]
        rbuf[0, s] = cs
        rbuf[1, s] = vs

    pltpu.sync_copy(rbuf.at[pl.ds(0, 2)], part_hbm.at[cid, :, pl.ds(goff, GSL)])


_pass1 = functools.partial(
    pl.kernel,
    mesh=_mesh,
    compiler_params=_params,
    out_type=jax.ShapeDtypeStruct((NC, 2, GP), jnp.float32),
    scratch_types=[
        pltpu.VMEM((CS,), jnp.float32),
        pltpu.VMEM((CS,), jnp.float32),
        pltpu.VMEM((CS,), jnp.int32),
        pltpu.VMEM((CS,), jnp.float32),
        pltpu.VMEM((CS,), jnp.float32),
        pltpu.VMEM((CS,), jnp.int32),
        pltpu.VMEM((2, GP), jnp.float32),
        pltpu.VMEM((2 * NS, GP // NS), jnp.float32),
        pltpu.VMEM_SHARED((2 * NS, GP), jnp.float32),
        pltpu.SemaphoreType.DMA,
        pltpu.SemaphoreType.DMA,
    ],
)(_p1_body)


def _p3_body(ch_hbm, lv_hbm, bi_hbm, part_hbm, formal_hbm, out_hbm,
             cb0, lb0, ib0, cb1, lb1, ib1, sbuf, fbuf, pbuf, smp, sm0, sm1):
    wid = _wid()
    base = wid * APW
    bufs = ((cb0, lb0, ib0, sm0), (cb1, lb1, ib1, sm1))

    def issue(k):
        cb, lb, ib, sem = bufs[k % 2]
        off = base + k * CS
        return (pltpu.async_copy(ch_hbm.at[pl.ds(off, CS)], cb, sem),
                pltpu.async_copy(lv_hbm.at[pl.ds(off, CS)], lb, sem),
                pltpu.async_copy(bi_hbm.at[pl.ds(off, CS)], ib, sem))

    part_cp = pltpu.async_copy(part_hbm, pbuf, smp)
    formal_cp = pltpu.async_copy(formal_hbm, fbuf, smp)
    pending = issue(0)
    part_cp.wait()
    formal_cp.wait()
    writeback = [None, None]

    for k in range(NCHUNK):
        for cp in pending:
            cp.wait()
        cb, lb, ib, sem = bufs[k % 2]
        if k + 1 < NCHUNK:
            wb = writeback[(k + 1) % 2]
            if wb is not None:
                wb.wait()
            pending = issue(k + 1)

        blk_lo = ib[pl.ds(0, L)][0] // L
        blk_hi = ib[pl.ds(CS - L, L)][L - 1] // L

        def scale_body(j, _):
            s = pl.ds(j * L, L)
            cs = pbuf[0, 0, s] + pbuf[1, 0, s]
            vs = pbuf[0, 1, s] + pbuf[1, 1, s]
            sbuf[s] = (fbuf[s] - cs) / (vs + EPS)
            return _

        lax.fori_loop(blk_lo, blk_hi + 1, scale_body, None)
        _exp_loop(lb)

        @plsc.parallel_loop(0, CS // L, unroll=UNROLL)
        def _(i):
            s = pl.ds(i * L, L)
            w = plsc.load_gather(sbuf, [ib[s]])
            cb[s] = cb[s] + lb[s] * w

        off = base + k * CS
        writeback[k % 2] = pltpu.async_copy(cb, out_hbm.at[pl.ds(off, CS)], sem)

    for wb in writeback:
        if wb is not None:
            wb.wait()


_pass3 = functools.partial(
    pl.kernel,
    mesh=_mesh,
    compiler_params=_params,
    out_type=jax.ShapeDtypeStruct((N,), jnp.float32),
    scratch_types=[
        pltpu.VMEM((CS,), jnp.float32),
        pltpu.VMEM((CS,), jnp.float32),
        pltpu.VMEM((CS,), jnp.int32),
        pltpu.VMEM((CS,), jnp.float32),
        pltpu.VMEM((CS,), jnp.float32),
        pltpu.VMEM((CS,), jnp.int32),
        pltpu.VMEM((GP,), jnp.float32),
        pltpu.VMEM((GP,), jnp.float32),
        pltpu.VMEM((NC, 2, GP), jnp.float32),
        pltpu.SemaphoreType.DMA,
        pltpu.SemaphoreType.DMA,
        pltpu.SemaphoreType.DMA,
    ],
)(_p3_body)


def kernel(charges, log_variance, batch_index, formal_charges):
    partials = _pass1(charges, log_variance, batch_index)
    formal_pad = jnp.pad(formal_charges.astype(jnp.float32), (0, GP - G))
    return _pass3(charges, log_variance, batch_index, partials, formal_pad)


# speedup vs baseline: 1.0986x; 1.0950x over previous
import functools

import jax
import jax.numpy as jnp
from jax import lax
from jax.experimental import pallas as pl
from jax.experimental.pallas import tpu as pltpu
from jax.experimental.pallas import tpu_sc as plsc

N = 1_600_000
G = 10_000
EPS = 1e-08

NC = 2
NS = 16
L = 16
NW = NC * NS
APW = N // NW
CS = 10_000
NCHUNK = APW // CS
GP = 10_240
GPW = GP // NW
UNROLL = 5

_mesh = plsc.VectorSubcoreMesh(core_axis_name="c", subcore_axis_name="s")
_params = pltpu.CompilerParams(
    needs_layout_passes=False, use_tc_tiling_on_sc=False
)


def _wid():
    return lax.axis_index("s") * NC + lax.axis_index("c")


def _p1_body(ch_hbm, lv_hbm, bi_hbm, part_hbm,
             cb0, lb0, ib0, cb1, lb1, ib1, accc, accv, sm0, sm1):
    wid = _wid()
    base = wid * APW
    bufs = ((cb0, lb0, ib0, sm0), (cb1, lb1, ib1, sm1))

    def issue(k):
        cb, lb, ib, sem = bufs[k % 2]
        off = base + k * CS
        return (pltpu.async_copy(ch_hbm.at[pl.ds(off, CS)], cb, sem),
                pltpu.async_copy(lv_hbm.at[pl.ds(off, CS)], lb, sem),
                pltpu.async_copy(bi_hbm.at[pl.ds(off, CS)], ib, sem))

    pending = issue(0)

    @plsc.parallel_loop(0, GP // L, unroll=8)
    def _(i):
        s = pl.ds(i * L, L)
        accc[s] = jnp.zeros((L,), jnp.float32)
        accv[s] = jnp.zeros((L,), jnp.float32)

    lanes = lax.iota(jnp.int32, L) * (CS // L)

    for k in range(NCHUNK):
        for cp in pending:
            cp.wait()
        cb, lb, ib, _ = bufs[k % 2]
        if k + 1 < NCHUNK:
            pending = issue(k + 1)

        @plsc.parallel_loop(0, CS // L, unroll=UNROLL)
        def _(i):
            pos = lanes + i
            idx = plsc.load_gather(ib, [pos])
            c = plsc.load_gather(cb, [pos])
            v = jnp.exp(plsc.load_gather(lb, [pos]))
            plsc.addupdate_scatter(accc, [idx], c)
            plsc.addupdate_scatter(accv, [idx], v)

    pltpu.sync_copy(accc, part_hbm.at[2 * wid])
    pltpu.sync_copy(accv, part_hbm.at[2 * wid + 1])


_pass1 = functools.partial(
    pl.kernel,
    mesh=_mesh,
    compiler_params=_params,
    out_type=jax.ShapeDtypeStruct((2 * NW, GP), jnp.float32),
    scratch_types=[
        pltpu.VMEM((CS,), jnp.float32),
        pltpu.VMEM((CS,), jnp.float32),
        pltpu.VMEM((CS,), jnp.int32),
        pltpu.VMEM((CS,), jnp.float32),
        pltpu.VMEM((CS,), jnp.float32),
        pltpu.VMEM((CS,), jnp.int32),
        pltpu.VMEM((GP,), jnp.float32),
        pltpu.VMEM((GP,), jnp.float32),
        pltpu.SemaphoreType.DMA,
        pltpu.SemaphoreType.DMA,
    ],
)(_p1_body)


def _p2_body(part_hbm, formal_hbm, scale_hbm, pbuf, fbuf, sbuf):
    wid = _wid()
    gbase = wid * GPW
    pltpu.sync_copy(part_hbm.at[:, pl.ds(gbase, GPW)], pbuf)
    pltpu.sync_copy(formal_hbm.at[pl.ds(gbase, GPW)], fbuf)

    def gbody(j, _):
        s = pl.ds(j * L, L)
        cs = jnp.zeros((L,), jnp.float32)
        vs = jnp.zeros((L,), jnp.float32)
        for t in range(NW):
            cs = cs + pbuf[2 * t, s]
            vs = vs + pbuf[2 * t + 1, s]
        sbuf[s] = (fbuf[s] - cs) / (vs + EPS)
        return _

    lax.fori_loop(0, GPW // L, gbody, None)
    pltpu.sync_copy(sbuf, scale_hbm.at[pl.ds(gbase, GPW)])


_pass2 = functools.partial(
    pl.kernel,
    mesh=_mesh,
    compiler_params=_params,
    out_type=jax.ShapeDtypeStruct((GP,), jnp.float32),
    scratch_types=[
        pltpu.VMEM((2 * NW, GPW), jnp.float32),
        pltpu.VMEM((GPW,), jnp.float32),
        pltpu.VMEM((GPW,), jnp.float32),
    ],
)(_p2_body)


def _p3_body(ch_hbm, lv_hbm, bi_hbm, scale_hbm, out_hbm,
             cb0, lb0, ib0, cb1, lb1, ib1, sbuf, sms, sm0, sm1):
    wid = _wid()
    base = wid * APW
    bufs = ((cb0, lb0, ib0, sm0), (cb1, lb1, ib1, sm1))

    def issue(k):
        cb, lb, ib, sem = bufs[k % 2]
        off = base + k * CS
        return (pltpu.async_copy(ch_hbm.at[pl.ds(off, CS)], cb, sem),
                pltpu.async_copy(lv_hbm.at[pl.ds(off, CS)], lb, sem),
                pltpu.async_copy(bi_hbm.at[pl.ds(off, CS)], ib, sem))

    scale_cp = pltpu.async_copy(scale_hbm, sbuf, sms)
    pending = issue(0)
    scale_cp.wait()
    writeback = [None, None]

    for k in range(NCHUNK):
        for cp in pending:
            cp.wait()
        cb, lb, ib, sem = bufs[k % 2]
        if k + 1 < NCHUNK:
            wb = writeback[(k + 1) % 2]
            if wb is not None:
                wb.wait()
            pending = issue(k + 1)

        @plsc.parallel_loop(0, CS // L, unroll=UNROLL)
        def _(i):
            s = pl.ds(i * L, L)
            w = plsc.load_gather(sbuf, [ib[s]])
            cb[s] = cb[s] + jnp.exp(lb[s]) * w

        off = base + k * CS
        writeback[k % 2] = pltpu.async_copy(cb, out_hbm.at[pl.ds(off, CS)], sem)

    for wb in writeback:
        if wb is not None:
            wb.wait()


_pass3 = functools.partial(
    pl.kernel,
    mesh=_mesh,
    compiler_params=_params,
    out_type=jax.ShapeDtypeStruct((N,), jnp.float32),
    scratch_types=[
        pltpu.VMEM((CS,), jnp.float32),
        pltpu.VMEM((CS,), jnp.float32),
        pltpu.VMEM((CS,), jnp.int32),
        pltpu.VMEM((CS,), jnp.float32),
        pltpu.VMEM((CS,), jnp.float32),
        pltpu.VMEM((CS,), jnp.int32),
        pltpu.VMEM((GP,), jnp.float32),
        pltpu.SemaphoreType.DMA,
        pltpu.SemaphoreType.DMA,
        pltpu.SemaphoreType.DMA,
    ],
)(_p3_body)


def kernel(charges, log_variance, batch_index, formal_charges):
    partials = _pass1(charges, log_variance, batch_index)
    formal_pad = jnp.pad(formal_charges.astype(jnp.float32), (0, GP - G))
    scale = _pass2(partials, formal_pad)
    return _pass3(charges, log_variance, batch_index, scale)
